# baseline (speedup 1.0000x reference)
import functools

import jax
import jax.numpy as jnp
from jax.experimental import pallas as pl
from jax.experimental.pallas import tpu as pltpu

_DN_T = (((1,), (1,)), ((), ()))
_DN_0 = (((0,), (0,)), ((), ()))


def _reg_kernel(*refs):
    part_ref = refs[-1]
    total = jnp.float32(0.0)
    for w_ref in refs[:-1]:
        w = w_ref[...]
        ones = jnp.ones((1, w.shape[1]), jnp.float32)
        ss = jax.lax.dot_general(ones, w * w, _DN_T,
                                 preferred_element_type=jnp.float32)
        reg = jnp.maximum(jnp.sqrt(ss) - 1.0, 0.0)
        total = total + jnp.sum(reg)
    part_ref[...] = jnp.full((1, 1, 128), total, dtype=jnp.float32)


def _dist(h_ref, t_ref, rw, idx_row, n_rels):
    tb = h_ref.shape[0]
    rows = jax.lax.broadcasted_iota(jnp.int16, (n_rels, tb), 0)
    oht = jnp.where(rows == idx_row, jnp.bfloat16(1.0), jnp.bfloat16(0.0))
    r = jax.lax.dot_general(oht, rw, _DN_0,
                            preferred_element_type=jnp.float32)
    d = h_ref[...] - t_ref[...] + r
    ones = jnp.ones((1, h_ref.shape[1]), jnp.float32)
    ss = jax.lax.dot_general(ones, d * d, _DN_T,
                             preferred_element_type=jnp.float32)
    return jnp.sqrt(ss)


def _gold_kernel(h_ref, t_ref, rw_ref, gi_ref, gold_ref, part_ref,
                 *, l2reg, n_rels):
    gold = _dist(h_ref, t_ref, rw_ref[...], gi_ref[0], n_rels)
    gold_ref[...] = gold[None]
    part_ref[...] = jnp.full((1, 1, 128), l2reg * jnp.sum(gold),
                             dtype=jnp.float32)


def _corrupt_kernel(h_ref, t_ref, rw_ref, bi_ref, gold_ref, part_ref,
                    *, margin, n_rels):
    corrupt = _dist(h_ref, t_ref, rw_ref[...], bi_ref[0], n_rels)
    contrib = jnp.maximum(margin + gold_ref[0] - corrupt, 0.0)
    part_ref[...] = jnp.full((1, 1, 128), jnp.sum(contrib),
                             dtype=jnp.float32)


def kernel(ents_w, rels_w, heads, rels, tails, heads_bad, rels_bad, tails_bad):
    margin, l2reg = 1.0, 0.1
    nEnts, dim = ents_w.shape
    nRels = rels_w.shape[0]
    ents_w = ents_w.astype(jnp.float32)
    B = int(heads.shape[0])

    n_streams = 8
    tile = 2048
    while nEnts % (tile * n_streams):
        tile //= 2
    n_steps = nEnts // (tile * n_streams)

    wspec = [pl.BlockSpec((tile, dim), functools.partial(
        lambda k, i: (i + k * n_steps, 0), k)) for k in range(n_streams)]
    reg_part = pl.pallas_call(
        _reg_kernel,
        out_shape=jax.ShapeDtypeStruct((n_steps, 1, 128), jnp.float32),
        grid=(n_steps,),
        in_specs=wspec,
        out_specs=pl.BlockSpec((1, 1, 128), lambda i: (i, 0, 0)),
        compiler_params=pltpu.CompilerParams(
            dimension_semantics=("parallel",)),
    )(*([ents_w] * n_streams))

    rels_bf = rels_w.astype(jnp.bfloat16)

    tb = 2048
    while B % tb:
        tb //= 2
    n_btiles = B // tb

    gi = jnp.asarray(rels, dtype=jnp.int16).reshape(n_btiles, 1, tb)
    bi = jnp.asarray(rels_bad, dtype=jnp.int16).reshape(n_btiles, 1, tb)

    gold_idx = jnp.concatenate([jnp.asarray(heads, jnp.int32),
                                jnp.asarray(tails, jnp.int32)])
    bad_idx = jnp.concatenate([jnp.asarray(heads_bad, jnp.int32),
                               jnp.asarray(tails_bad, jnp.int32)])
    gslab = ents_w.at[gold_idx].get(mode="promise_in_bounds")
    bslab = ents_w.at[bad_idx].get(mode="promise_in_bounds")

    espec = [pl.BlockSpec((tb, dim), functools.partial(
        lambda k, i: (i + k * n_btiles, 0), k)) for k in range(2)]
    rw_spec = pl.BlockSpec((nRels, dim), lambda i: (0, 0))
    idx_spec = pl.BlockSpec((1, 1, tb), lambda i: (i, 0, 0))
    vec_spec = pl.BlockSpec((1, 1, tb), lambda i: (i, 0, 0))
    part_spec = pl.BlockSpec((1, 1, 128), lambda i: (i, 0, 0))

    gold_vec, gold_part = pl.pallas_call(
        functools.partial(_gold_kernel, l2reg=l2reg, n_rels=nRels),
        out_shape=(jax.ShapeDtypeStruct((n_btiles, 1, tb), jnp.float32),
                   jax.ShapeDtypeStruct((n_btiles, 1, 128), jnp.float32)),
        grid=(n_btiles,),
        in_specs=espec + [rw_spec, idx_spec],
        out_specs=(vec_spec, part_spec),
        compiler_params=pltpu.CompilerParams(
            dimension_semantics=("parallel",)),
    )(gslab, gslab, rels_bf, gi)

    loss_part = pl.pallas_call(
        functools.partial(_corrupt_kernel, margin=margin, n_rels=nRels),
        out_shape=jax.ShapeDtypeStruct((n_btiles, 1, 128), jnp.float32),
        grid=(n_btiles,),
        in_specs=espec + [rw_spec, idx_spec, vec_spec],
        out_specs=part_spec,
        compiler_params=pltpu.CompilerParams(
            dimension_semantics=("parallel",)),
    )(bslab, bslab, rels_bf, bi, gold_vec)

    return (jnp.sum(loss_part[:, 0, 0]) + jnp.sum(gold_part[:, 0, 0])
            + l2reg * jnp.sum(reg_part[:, 0, 0]))


# speedup vs baseline: 1.0425x; 1.0425x over previous
import functools

import jax
import jax.numpy as jnp
from jax.experimental import pallas as pl
from jax.experimental.pallas import tpu as pltpu

_DN_T = (((1,), (1,)), ((), ()))
_DN_0 = (((0,), (0,)), ((), ()))


def _reg_kernel(*refs):
    part_ref = refs[-1]
    total = jnp.float32(0.0)
    for w_ref in refs[:-1]:
        w = w_ref[...]
        ones = jnp.ones((1, w.shape[1]), jnp.float32)
        ss = jax.lax.dot_general(ones, w * w, _DN_T,
                                 preferred_element_type=jnp.float32)
        reg = jnp.maximum(jnp.sqrt(ss) - 1.0, 0.0)
        total = total + jnp.sum(reg)
    part_ref[...] = jnp.full((1, 1, 128), total, dtype=jnp.float32)


def _loss_kernel(gh_ref, gt_ref, bh_ref, bt_ref, rw_ref, gi_ref, bi_ref,
                 part_ref, *, margin, l2reg, n_rels):
    tb = gh_ref.shape[0]
    dim = gh_ref.shape[1]
    rw = rw_ref[...]
    rows = jax.lax.broadcasted_iota(jnp.int16, (n_rels, tb), 0)
    ones = jnp.ones((1, dim), jnp.float32)
    one_b = jnp.bfloat16(1.0)
    zero_b = jnp.bfloat16(0.0)

    g_oht = jnp.where(rows == gi_ref[0], one_b, zero_b)
    gr = jax.lax.dot_general(g_oht, rw, _DN_0,
                             preferred_element_type=jnp.float32)
    gd = gh_ref[...] - gt_ref[...] + gr
    ssg = jax.lax.dot_general(ones, gd * gd, _DN_T,
                              preferred_element_type=jnp.float32)
    gold = jnp.sqrt(ssg)

    b_oht = jnp.where(rows == bi_ref[0], one_b, zero_b)
    br = jax.lax.dot_general(b_oht, rw, _DN_0,
                             preferred_element_type=jnp.float32)
    bd = bh_ref[...] - bt_ref[...] + br
    ssb = jax.lax.dot_general(ones, bd * bd, _DN_T,
                              preferred_element_type=jnp.float32)
    corrupt = jnp.sqrt(ssb)

    contrib = jnp.maximum(margin + gold - corrupt, 0.0) + l2reg * gold
    part_ref[...] = jnp.full((1, 1, 128), jnp.sum(contrib), dtype=jnp.float32)


def _loss_chunk(eslab, rels_bf, gi, bi, tile0, tb, dim, n_rels, margin, l2reg):
    rows = eslab.shape[0] // 4
    n_btiles = rows // tb
    off = rows // tb

    espec = [pl.BlockSpec((tb, dim), functools.partial(
        lambda k, i: (i + k * off, 0), k)) for k in range(4)]
    rw_spec = pl.BlockSpec((n_rels, dim), lambda i: (0, 0))
    idx_spec = pl.BlockSpec((1, 1, tb), lambda i: (i + tile0, 0, 0))

    return pl.pallas_call(
        functools.partial(_loss_kernel, margin=margin, l2reg=l2reg,
                          n_rels=n_rels),
        out_shape=jax.ShapeDtypeStruct((n_btiles, 1, 128), jnp.float32),
        grid=(n_btiles,),
        in_specs=espec + [rw_spec, idx_spec, idx_spec],
        out_specs=pl.BlockSpec((1, 1, 128), lambda i: (i, 0, 0)),
        compiler_params=pltpu.CompilerParams(
            dimension_semantics=("parallel",)),
    )(eslab, eslab, eslab, eslab, rels_bf, gi, bi)


def kernel(ents_w, rels_w, heads, rels, tails, heads_bad, rels_bad, tails_bad):
    margin, l2reg = 1.0, 0.1
    nEnts, dim = ents_w.shape
    nRels = rels_w.shape[0]
    ents_w = ents_w.astype(jnp.float32)
    B = int(heads.shape[0])

    n_streams = 8
    tile = 2048
    while nEnts % (tile * n_streams):
        tile //= 2
    n_steps = nEnts // (tile * n_streams)

    wspec = [pl.BlockSpec((tile, dim), functools.partial(
        lambda k, i: (i + k * n_steps, 0), k)) for k in range(n_streams)]
    reg_part = pl.pallas_call(
        _reg_kernel,
        out_shape=jax.ShapeDtypeStruct((n_steps, 1, 128), jnp.float32),
        grid=(n_steps,),
        in_specs=wspec,
        out_specs=pl.BlockSpec((1, 1, 128), lambda i: (i, 0, 0)),
        compiler_params=pltpu.CompilerParams(
            dimension_semantics=("parallel",)),
    )(*([ents_w] * n_streams))

    rels_bf = rels_w.astype(jnp.bfloat16)

    idx = [jnp.asarray(a, dtype=jnp.int32)
           for a in (heads, tails, heads_bad, tails_bad)]

    tb = 2048
    while (B // 2) % tb:
        tb //= 2

    cut = (B * 3 // 8) // tb * tb
    bounds = [(0, cut), (cut, B)]

    eidx = jnp.concatenate([a[lo:hi] for (lo, hi) in bounds for a in idx])
    gi = jnp.asarray(rels, dtype=jnp.int16).reshape(B // tb, 1, tb)
    bi = jnp.asarray(rels_bad, dtype=jnp.int16).reshape(B // tb, 1, tb)

    parts = []
    base = 0
    for lo, hi in bounds:
        n = hi - lo
        eslab = ents_w.at[
            jax.lax.slice_in_dim(eidx, base, base + 4 * n)
        ].get(mode="promise_in_bounds")
        parts.append(_loss_chunk(eslab, rels_bf, gi, bi, lo // tb, tb, dim,
                                 nRels, margin, l2reg))
        base += 4 * n

    return (jnp.sum(parts[0][:, 0, 0]) + jnp.sum(parts[1][:, 0, 0])
            + l2reg * jnp.sum(reg_part[:, 0, 0]))


# speedup vs baseline: 1.0718x; 1.0281x over previous
import functools

import jax
import jax.numpy as jnp
from jax.experimental import pallas as pl
from jax.experimental.pallas import tpu as pltpu

_DN_T = (((1,), (1,)), ((), ()))
_DN_0 = (((0,), (0,)), ((), ()))


def _reg_kernel(*refs):
    part_ref = refs[-1]
    total = jnp.float32(0.0)
    for w_ref in refs[:-1]:
        w = w_ref[...]
        ones = jnp.ones((1, w.shape[1]), jnp.float32)
        ss = jax.lax.dot_general(ones, w * w, _DN_T,
                                 preferred_element_type=jnp.float32)
        reg = jnp.maximum(jnp.sqrt(ss) - 1.0, 0.0)
        total = total + jnp.sum(reg)
    part_ref[...] = jnp.full((1, 1, 128), total, dtype=jnp.float32)


def _loss_kernel(gh_ref, gt_ref, bh_ref, bt_ref, rw_ref, gi_ref, bi_ref,
                 part_ref, *, margin, l2reg, n_rels):
    tb = gh_ref.shape[0]
    dim = gh_ref.shape[1]
    rw = rw_ref[...]
    rows = jax.lax.broadcasted_iota(jnp.int16, (n_rels, tb), 0)
    ones = jnp.ones((1, dim), jnp.float32)
    one_b = jnp.bfloat16(1.0)
    zero_b = jnp.bfloat16(0.0)

    g_oht = jnp.where(rows == gi_ref[0], one_b, zero_b)
    gr = jax.lax.dot_general(g_oht, rw, _DN_0,
                             preferred_element_type=jnp.float32)
    gd = gh_ref[...] - gt_ref[...] + gr
    ssg = jax.lax.dot_general(ones, gd * gd, _DN_T,
                              preferred_element_type=jnp.float32)
    gold = jnp.sqrt(ssg)

    b_oht = jnp.where(rows == bi_ref[0], one_b, zero_b)
    br = jax.lax.dot_general(b_oht, rw, _DN_0,
                             preferred_element_type=jnp.float32)
    bd = bh_ref[...] - bt_ref[...] + br
    ssb = jax.lax.dot_general(ones, bd * bd, _DN_T,
                              preferred_element_type=jnp.float32)
    corrupt = jnp.sqrt(ssb)

    contrib = jnp.maximum(margin + gold - corrupt, 0.0) + l2reg * gold
    part_ref[...] = jnp.full((1, 1, 128), jnp.sum(contrib), dtype=jnp.float32)


def _loss_chunk(eslab, rels_bf, gi, bi, tile0, tb, dim, n_rels, margin, l2reg):
    rows = eslab.shape[0] // 4
    n_btiles = rows // tb
    off = rows // tb

    espec = [pl.BlockSpec((tb, dim), functools.partial(
        lambda k, i: (i + k * off, 0), k)) for k in range(4)]
    rw_spec = pl.BlockSpec((n_rels, dim), lambda i: (0, 0))
    idx_spec = pl.BlockSpec((1, 1, tb), lambda i: (i + tile0, 0, 0))

    return pl.pallas_call(
        functools.partial(_loss_kernel, margin=margin, l2reg=l2reg,
                          n_rels=n_rels),
        out_shape=jax.ShapeDtypeStruct((n_btiles, 1, 128), jnp.float32),
        grid=(n_btiles,),
        in_specs=espec + [rw_spec, idx_spec, idx_spec],
        out_specs=pl.BlockSpec((1, 1, 128), lambda i: (i, 0, 0)),
        compiler_params=pltpu.CompilerParams(
            dimension_semantics=("parallel",)),
    )(eslab, eslab, eslab, eslab, rels_bf, gi, bi)


def kernel(ents_w, rels_w, heads, rels, tails, heads_bad, rels_bad, tails_bad):
    margin, l2reg = 1.0, 0.1
    nEnts, dim = ents_w.shape
    nRels = rels_w.shape[0]
    ents_w = ents_w.astype(jnp.float32)
    B = int(heads.shape[0])

    n_streams = 8
    tile = 2048
    while nEnts % (tile * n_streams):
        tile //= 2
    n_steps = nEnts // (tile * n_streams)

    wspec = [pl.BlockSpec((tile, dim), functools.partial(
        lambda k, i: (i + k * n_steps, 0), k)) for k in range(n_streams)]
    reg_part = pl.pallas_call(
        _reg_kernel,
        out_shape=jax.ShapeDtypeStruct((n_steps, 1, 128), jnp.float32),
        grid=(n_steps,),
        in_specs=wspec,
        out_specs=pl.BlockSpec((1, 1, 128), lambda i: (i, 0, 0)),
        compiler_params=pltpu.CompilerParams(
            dimension_semantics=("parallel",)),
    )(*([ents_w] * n_streams))

    rels_bf = rels_w.astype(jnp.bfloat16)

    idx = [jnp.asarray(a, dtype=jnp.int32)
           for a in (heads, tails, heads_bad, tails_bad)]

    tb = 4096
    while (B // 2) % tb:
        tb //= 2

    cut = min(max((B * 3 // 8) // tb * tb, tb), B - tb)
    bounds = [(0, cut), (cut, B)]

    eidx = jnp.concatenate([a[lo:hi] for (lo, hi) in bounds for a in idx])
    gi = jnp.asarray(rels, dtype=jnp.int16).reshape(B // tb, 1, tb)
    bi = jnp.asarray(rels_bad, dtype=jnp.int16).reshape(B // tb, 1, tb)

    parts = []
    base = 0
    for lo, hi in bounds:
        n = hi - lo
        eslab = ents_w.at[
            jax.lax.slice_in_dim(eidx, base, base + 4 * n)
        ].get(mode="promise_in_bounds")
        parts.append(_loss_chunk(eslab, rels_bf, gi, bi, lo // tb, tb, dim,
                                 nRels, margin, l2reg))
        base += 4 * n

    return (jnp.sum(parts[0][:, 0, 0]) + jnp.sum(parts[1][:, 0, 0])
            + l2reg * jnp.sum(reg_part[:, 0, 0]))
